# Initial kernel scaffold; baseline (speedup 1.0000x reference)
import jax
import jax.numpy as jnp
from jax.experimental import pallas as pl


def kernel(point_features, point_coords, batch_size, W0, b0, g0, be0, W1, b1, g1, be1, Wo, bo, go, beo):
    raise NotImplementedError("write your pallas kernel here")


# speedup vs baseline: 9.1515x; 9.1515x over previous
import functools

import jax
import jax.numpy as jnp
from jax import lax
from jax.experimental import pallas as pl
from jax.experimental.pallas import tpu as pltpu
from jax.experimental.pallas import tpu_sc as plsc

_N = 4096
_K = 16
_CW = 256
_NCH = _N // _CW
_T = 256
_TE = 32
_IP = False

_NC = 2
_NS = 16
_NW = _NC * _NS


def _knn_body(x_ref, xt_ref, out_ref):
    b = pl.program_id(0)
    xr = x_ref[0]
    xa = xt_ref[0]
    inner = lax.dot_general(xr, xa, (((1,), (0,)), ((), ())),
                            preferred_element_type=jnp.float32)
    sqr = jnp.sum(xr * xr, axis=1, keepdims=True)
    sqa = jnp.sum(xa * xa, axis=0, keepdims=True)
    d = sqr - 2.0 * inner + sqa
    d = jnp.maximum(d, 0.0) + 1.0
    bits = lax.bitcast_convert_type(d, jnp.int32)
    lane = lax.broadcasted_iota(jnp.int32, d.shape, 1)
    keys = lax.bitcast_convert_type((bits & ~jnp.int32(255)) | (lane & 255),
                                    jnp.float32)
    cands = []
    for c in range(_NCH):
        kc = keys[:, c * _CW:(c + 1) * _CW]
        cur = jnp.zeros((kc.shape[0], 1), jnp.float32)
        for _ in range(_K):
            m = jnp.min(jnp.where(kc > cur, kc, jnp.inf), axis=1, keepdims=True)
            cands.append(m)
            cur = m
    ck = jnp.concatenate(cands, axis=1)
    cbits = lax.bitcast_convert_type(ck, jnp.int32)
    lane2 = lax.broadcasted_iota(jnp.int32, ck.shape, 1)
    candcol = ((cbits & 255) + (lane2 // _K) * _CW).astype(jnp.float32)
    for t in range(_K):
        m = jnp.min(ck, axis=1, keepdims=True)
        eq = ck == m
        j = jnp.min(jnp.where(eq, candcol, jnp.float32(1e9)), axis=1)
        out_ref[0, t, :] = j.astype(jnp.int32) + b * _N
        ck = jnp.where(candcol == j[:, None], jnp.inf, ck)


def _knn_topk(x, xt):
    bsz, n, c = x.shape
    return pl.pallas_call(
        _knn_body,
        grid=(bsz, n // _T),
        in_specs=[pl.BlockSpec((1, _T, c), lambda b, r: (b, r, 0)),
                  pl.BlockSpec((1, c, n), lambda b, r: (b, 0, 0))],
        out_specs=pl.BlockSpec((1, _K, _T), lambda b, r: (b, 0, r)),
        out_shape=jax.ShapeDtypeStruct((bsz, _K, n), jnp.int32),
        interpret=_IP,
    )(x, xt)


def _sc_feat(x, idx_flat):
    bn, c = x.shape
    pw = bn // _NW
    cp = 8
    rows_n = cp * _K
    ng = c // 16
    mesh = plsc.VectorSubcoreMesh(core_axis_name="c", subcore_axis_name="s")

    @functools.partial(
        pl.kernel,
        out_type=jax.ShapeDtypeStruct((bn * _K, c), jnp.float32),
        mesh=mesh,
        scratch_types=[pltpu.VMEM((rows_n,), jnp.int32),
                       pltpu.VMEM((rows_n, c), jnp.float32),
                       pltpu.VMEM((cp, c), jnp.float32),
                       pltpu.SemaphoreType.DMA],
        compiler_params=pltpu.CompilerParams(use_tc_tiling_on_sc=False),
    )
    def run(x_hbm, idx_hbm, feat_hbm, idxb, rows, xib, sem):
        wid = lax.axis_index("s") * _NC + lax.axis_index("c")
        base = wid * pw

        def chunk(ci, carry):
            pb = base + ci * cp
            pltpu.sync_copy(idx_hbm.at[pl.ds(pb * _K, rows_n)], idxb)
            pltpu.async_copy(x_hbm.at[idxb], rows, sem).wait()
            pltpu.sync_copy(x_hbm.at[pl.ds(pb, cp)], xib)
            for p in range(cp):
                for g in range(ng):
                    sl = pl.ds(g * 16, 16)
                    xv = xib[p, sl]
                    for rr in range(_K):
                        rows[p * _K + rr, sl] = rows[p * _K + rr, sl] - xv
            pltpu.sync_copy(rows, feat_hbm.at[pl.ds(pb * _K, rows_n)])
            return carry

        lax.fori_loop(0, pw // cp, chunk, 0)

    return run(x, idx_flat)


def _edge_body(feat_ref, x_ref, wr_ref, wl_ref, b_ref, omax_ref, acc_ref):
    @pl.when(pl.program_id(0) == 0)
    def _():
        acc_ref[...] = jnp.zeros_like(acc_ref)

    ydiff = lax.dot_general(feat_ref[...], wr_ref[...], (((1,), (0,)), ((), ())),
                            preferred_element_type=jnp.float32)
    z = lax.dot_general(x_ref[...], wl_ref[...], (((1,), (0,)), ((), ())),
                        preferred_element_type=jnp.float32) + b_ref[...]
    ts = jnp.zeros((1, 64), jnp.float32)
    tq = jnp.zeros((1, 64), jnp.float32)
    for n in range(_TE):
        slab = ydiff[n * _K:(n + 1) * _K, :] + z[n:n + 1, :]
        omax_ref[n:n + 1, :] = jnp.max(slab, axis=0, keepdims=True)
        ts = ts + jnp.sum(slab, axis=0, keepdims=True)
        tq = tq + jnp.sum(slab * slab, axis=0, keepdims=True)
    acc_ref[...] += jnp.concatenate([ts, tq, jnp.zeros((6, 64), jnp.float32)],
                                    axis=0)


def _edge(feat, x, wr, wl, bvec):
    bn, c = x.shape
    return pl.pallas_call(
        _edge_body,
        grid=(bn // _TE,),
        in_specs=[pl.BlockSpec((_TE * _K, c), lambda i: (i, 0)),
                  pl.BlockSpec((_TE, c), lambda i: (i, 0)),
                  pl.BlockSpec((c, 64), lambda i: (0, 0)),
                  pl.BlockSpec((c, 64), lambda i: (0, 0)),
                  pl.BlockSpec((1, 64), lambda i: (0, 0))],
        out_specs=[pl.BlockSpec((_TE, 64), lambda i: (i, 0)),
                   pl.BlockSpec((8, 64), lambda i: (0, 0))],
        out_shape=[jax.ShapeDtypeStruct((bn, 64), jnp.float32),
                   jax.ShapeDtypeStruct((8, 64), jnp.float32)],
        interpret=_IP,
    )(feat, x, wr, wl, bvec)


def _norm_body(my_ref, acc_ref, g_ref, be_ref, o_ref, *, edges):
    acc = acc_ref[...]
    cnt = jnp.float32(edges)
    mean = acc[0:1] / cnt
    var = acc[1:2] / cnt - mean * mean
    yn = (my_ref[...] - mean) / jnp.sqrt(var + 1e-5)
    o_ref[...] = jnp.maximum(g_ref[...] * yn + be_ref[...], 0.0)


def _normalize(my, acc, g, be):
    bn = my.shape[0]
    tp = 2048
    return pl.pallas_call(
        functools.partial(_norm_body, edges=bn * _K),
        grid=(bn // tp,),
        in_specs=[pl.BlockSpec((tp, 64), lambda i: (i, 0)),
                  pl.BlockSpec((8, 64), lambda i: (0, 0)),
                  pl.BlockSpec((1, 64), lambda i: (0, 0)),
                  pl.BlockSpec((1, 64), lambda i: (0, 0))],
        out_specs=pl.BlockSpec((tp, 64), lambda i: (i, 0)),
        out_shape=jax.ShapeDtypeStruct((bn, 64), jnp.float32),
        interpret=_IP,
    )(my, acc, g, be)


def _mom_body(x_ref, m_ref, s_ref):
    @pl.when(pl.program_id(0) == 0)
    def _():
        m_ref[...] = jnp.zeros_like(m_ref)
        s_ref[...] = jnp.zeros_like(s_ref)

    x = x_ref[...]
    m_ref[...] += lax.dot_general(x, x, (((0,), (0,)), ((), ())),
                                  preferred_element_type=jnp.float32)
    colsum = jnp.sum(x, axis=0, keepdims=True)
    s_ref[...] += jnp.concatenate([colsum] + [jnp.zeros_like(colsum)] * 7, axis=0)


def _moments(x):
    bn = x.shape[0]
    tp = 2048
    return pl.pallas_call(
        _mom_body,
        grid=(bn // tp,),
        in_specs=[pl.BlockSpec((tp, 64), lambda i: (i, 0))],
        out_specs=[pl.BlockSpec((64, 64), lambda i: (0, 0)),
                   pl.BlockSpec((8, 64), lambda i: (0, 0))],
        out_shape=[jax.ShapeDtypeStruct((64, 64), jnp.float32),
                   jax.ShapeDtypeStruct((8, 64), jnp.float32)],
        interpret=_IP,
    )(x)


def _final_body(x_ref, wt_ref, bo_ref, m_ref, s_ref, go_ref, beo_ref, o_ref, *,
                bn_total):
    bnf = jnp.float32(bn_total)
    meanx = s_ref[0:1] / bnf
    wt = wt_ref[...]
    mt = lax.dot_general(meanx, wt, (((1,), (0,)), ((), ())),
                         preferred_element_type=jnp.float32) + bo_ref[...]
    cov = m_ref[...] / bnf - lax.dot_general(meanx, meanx, (((0,), (0,)), ((), ())),
                                             preferred_element_type=jnp.float32)
    cw = lax.dot_general(cov, wt, (((1,), (0,)), ((), ())),
                         preferred_element_type=jnp.float32)
    var = jnp.sum(cw * wt, axis=0, keepdims=True)
    a = go_ref[...] / jnp.sqrt(var + 1e-5)
    d = beo_ref[...] - a * mt
    wa = wt * a
    o_ref[...] = jnp.maximum(
        lax.dot_general(x_ref[...], wa, (((1,), (0,)), ((), ())),
                        preferred_element_type=jnp.float32) + d, 0.0)


def _final(x, wt, bo, m, s, go, beo):
    bn = x.shape[0]
    tp = 2048
    return pl.pallas_call(
        functools.partial(_final_body, bn_total=bn),
        grid=(bn // tp,),
        in_specs=[pl.BlockSpec((tp, 64), lambda i: (i, 0)),
                  pl.BlockSpec((64, 128), lambda i: (0, 0)),
                  pl.BlockSpec((1, 128), lambda i: (0, 0)),
                  pl.BlockSpec((64, 64), lambda i: (0, 0)),
                  pl.BlockSpec((8, 64), lambda i: (0, 0)),
                  pl.BlockSpec((1, 128), lambda i: (0, 0)),
                  pl.BlockSpec((1, 128), lambda i: (0, 0))],
        out_specs=pl.BlockSpec((tp, 128), lambda i: (i, 0)),
        out_shape=jax.ShapeDtypeStruct((bn, 128), jnp.float32),
        interpret=_IP,
    )(x, wt, bo, m, s, go, beo)


def _edge_layer(x_pad, c_real, idx_flat, w, b, g, be):
    cp = x_pad.shape[1]
    wl = jnp.zeros((cp, 64), jnp.float32).at[:c_real].set(jnp.transpose(w[:, :c_real]))
    wr = jnp.zeros((cp, 64), jnp.float32).at[:c_real].set(jnp.transpose(w[:, c_real:]))
    feat = _sc_feat(x_pad, idx_flat)
    my, acc = _edge(feat, x_pad, wr, wl, b.reshape(1, 64))
    return _normalize(my, acc, g.reshape(1, 64), be.reshape(1, 64))


def kernel(point_features, point_coords, batch_size, W0, b0, g0, be0,
           W1, b1, g1, be1, Wo, bo, go, beo):
    del batch_size
    bn = point_features.shape[0]
    bsz = bn // _N

    coords = point_coords[:, 1:]
    xf = jnp.concatenate([coords, point_features], axis=-1)
    xf_pad = jnp.pad(xf, ((0, 0), (0, 32 - xf.shape[1])))
    pos_rows = jnp.pad(coords.reshape(bsz, _N, 3), ((0, 0), (0, 0), (0, 5)))
    idx1 = _knn_topk(pos_rows, jnp.transpose(pos_rows, (0, 2, 1)))
    idx1f = jnp.transpose(idx1, (0, 2, 1)).reshape(-1)
    x0 = _edge_layer(xf_pad, xf.shape[1], idx1f, W0, b0, g0, be0)

    x0r = x0.reshape(bsz, _N, 64)
    idx2 = _knn_topk(x0r, jnp.transpose(x0r, (0, 2, 1)))
    idx2f = jnp.transpose(idx2, (0, 2, 1)).reshape(-1)
    x1 = _edge_layer(x0, 64, idx2f, W1, b1, g1, be1)

    m, s = _moments(x1)
    return _final(x1, jnp.transpose(Wo), bo.reshape(1, 128), m, s,
                  go.reshape(1, 128), beo.reshape(1, 128))


# speedup vs baseline: 13.8665x; 1.5152x over previous
import functools

import jax
import jax.numpy as jnp
from jax import lax
from jax.experimental import pallas as pl
from jax.experimental.pallas import tpu as pltpu
from jax.experimental.pallas import tpu_sc as plsc

_N = 4096
_K = 16
_CW = 128
_NCH = _N // _CW
_T = 256
_TE = 32
_IP = False

_NC = 2
_NS = 16
_NW = _NC * _NS


def _knn_body(x_ref, xt_ref, out_ref):
    b = pl.program_id(0)
    xa = x_ref[0]
    xq = xt_ref[0]
    inner = lax.dot_general(xa, xq, (((1,), (0,)), ((), ())),
                            preferred_element_type=jnp.float32)
    sqa = jnp.sum(xa * xa, axis=1, keepdims=True)
    sqq = jnp.sum(xq * xq, axis=0, keepdims=True)
    d = (sqq - 2.0 * inner) + sqa
    d = jnp.maximum(d, 0.0) + 1.0
    bits = lax.bitcast_convert_type(d, jnp.int32)
    row = lax.broadcasted_iota(jnp.int32, d.shape, 0)
    keys = lax.bitcast_convert_type((bits & ~jnp.int32(_CW - 1)) | (row & (_CW - 1)),
                                    jnp.float32)
    cands = []
    for c in range(_NCH):
        kc = keys[c * _CW:(c + 1) * _CW, :]
        cur = jnp.zeros((1, kc.shape[1]), jnp.float32)
        for _ in range(_K):
            m = jnp.min(jnp.where(kc > cur, kc, jnp.inf), axis=0, keepdims=True)
            cands.append(m)
            cur = m
    ck = jnp.concatenate(cands, axis=0)
    cbits = lax.bitcast_convert_type(ck, jnp.int32)
    row2 = lax.broadcasted_iota(jnp.int32, ck.shape, 0)
    candcol = ((cbits & (_CW - 1)) + (row2 // _K) * _CW).astype(jnp.float32)
    for t in range(_K):
        m = jnp.min(ck, axis=0, keepdims=True)
        eq = ck == m
        j = jnp.min(jnp.where(eq, candcol, jnp.float32(1e9)), axis=0)
        out_ref[0, t, :] = j.astype(jnp.int32) + b * _N
        ck = jnp.where(candcol == j[None, :], jnp.inf, ck)


def _knn_topk(x, xt):
    bsz, n, c = x.shape
    return pl.pallas_call(
        _knn_body,
        grid=(bsz, n // _T),
        in_specs=[pl.BlockSpec((1, n, c), lambda b, r: (b, 0, 0)),
                  pl.BlockSpec((1, c, _T), lambda b, r: (b, 0, r))],
        out_specs=pl.BlockSpec((1, _K, _T), lambda b, r: (b, 0, r)),
        out_shape=jax.ShapeDtypeStruct((bsz, _K, n), jnp.int32),
        interpret=_IP,
    )(x, xt)


def _sc_feat(x, idx_flat):
    bn, c = x.shape
    pw = bn // _NW
    cp = 8
    rows_n = cp * _K
    ng = c // 16
    mesh = plsc.VectorSubcoreMesh(core_axis_name="c", subcore_axis_name="s")

    @functools.partial(
        pl.kernel,
        out_type=jax.ShapeDtypeStruct((bn * _K, c), jnp.float32),
        mesh=mesh,
        scratch_types=[pltpu.VMEM((rows_n,), jnp.int32),
                       pltpu.VMEM((rows_n, c), jnp.float32),
                       pltpu.VMEM((cp, c), jnp.float32),
                       pltpu.SemaphoreType.DMA],
        compiler_params=pltpu.CompilerParams(use_tc_tiling_on_sc=False),
    )
    def run(x_hbm, idx_hbm, feat_hbm, idxb, rows, xib, sem):
        wid = lax.axis_index("s") * _NC + lax.axis_index("c")
        base = wid * pw

        def chunk(ci, carry):
            pb = base + ci * cp
            pltpu.sync_copy(idx_hbm.at[pl.ds(pb * _K, rows_n)], idxb)
            pltpu.async_copy(x_hbm.at[idxb], rows, sem).wait()
            pltpu.sync_copy(x_hbm.at[pl.ds(pb, cp)], xib)
            for p in range(cp):
                for g in range(ng):
                    sl = pl.ds(g * 16, 16)
                    xv = xib[p, sl]
                    for rr in range(_K):
                        rows[p * _K + rr, sl] = rows[p * _K + rr, sl] - xv
            pltpu.sync_copy(rows, feat_hbm.at[pl.ds(pb * _K, rows_n)])
            return carry

        lax.fori_loop(0, pw // cp, chunk, 0)

    return run(x, idx_flat)


def _edge_body(feat_ref, x_ref, wr_ref, wl_ref, b_ref, omax_ref, acc_ref):
    @pl.when(pl.program_id(0) == 0)
    def _():
        acc_ref[...] = jnp.zeros_like(acc_ref)

    ydiff = lax.dot_general(feat_ref[...], wr_ref[...], (((1,), (0,)), ((), ())),
                            preferred_element_type=jnp.float32)
    z = lax.dot_general(x_ref[...], wl_ref[...], (((1,), (0,)), ((), ())),
                        preferred_element_type=jnp.float32) + b_ref[...]
    ts = jnp.zeros((1, 64), jnp.float32)
    tq = jnp.zeros((1, 64), jnp.float32)
    for n in range(_TE):
        slab = ydiff[n * _K:(n + 1) * _K, :] + z[n:n + 1, :]
        omax_ref[n:n + 1, :] = jnp.max(slab, axis=0, keepdims=True)
        ts = ts + jnp.sum(slab, axis=0, keepdims=True)
        tq = tq + jnp.sum(slab * slab, axis=0, keepdims=True)
    acc_ref[...] += jnp.concatenate([ts, tq, jnp.zeros((6, 64), jnp.float32)],
                                    axis=0)


def _edge(feat, x, wr, wl, bvec):
    bn, c = x.shape
    return pl.pallas_call(
        _edge_body,
        grid=(bn // _TE,),
        in_specs=[pl.BlockSpec((_TE * _K, c), lambda i: (i, 0)),
                  pl.BlockSpec((_TE, c), lambda i: (i, 0)),
                  pl.BlockSpec((c, 64), lambda i: (0, 0)),
                  pl.BlockSpec((c, 64), lambda i: (0, 0)),
                  pl.BlockSpec((1, 64), lambda i: (0, 0))],
        out_specs=[pl.BlockSpec((_TE, 64), lambda i: (i, 0)),
                   pl.BlockSpec((8, 64), lambda i: (0, 0))],
        out_shape=[jax.ShapeDtypeStruct((bn, 64), jnp.float32),
                   jax.ShapeDtypeStruct((8, 64), jnp.float32)],
        interpret=_IP,
    )(feat, x, wr, wl, bvec)


def _norm_body(my_ref, acc_ref, g_ref, be_ref, o_ref, *, edges):
    acc = acc_ref[...]
    cnt = jnp.float32(edges)
    mean = acc[0:1] / cnt
    var = acc[1:2] / cnt - mean * mean
    yn = (my_ref[...] - mean) / jnp.sqrt(var + 1e-5)
    o_ref[...] = jnp.maximum(g_ref[...] * yn + be_ref[...], 0.0)


def _normalize(my, acc, g, be):
    bn = my.shape[0]
    tp = 2048
    return pl.pallas_call(
        functools.partial(_norm_body, edges=bn * _K),
        grid=(bn // tp,),
        in_specs=[pl.BlockSpec((tp, 64), lambda i: (i, 0)),
                  pl.BlockSpec((8, 64), lambda i: (0, 0)),
                  pl.BlockSpec((1, 64), lambda i: (0, 0)),
                  pl.BlockSpec((1, 64), lambda i: (0, 0))],
        out_specs=pl.BlockSpec((tp, 64), lambda i: (i, 0)),
        out_shape=jax.ShapeDtypeStruct((bn, 64), jnp.float32),
        interpret=_IP,
    )(my, acc, g, be)


def _mom_body(x_ref, m_ref, s_ref):
    @pl.when(pl.program_id(0) == 0)
    def _():
        m_ref[...] = jnp.zeros_like(m_ref)
        s_ref[...] = jnp.zeros_like(s_ref)

    x = x_ref[...]
    m_ref[...] += lax.dot_general(x, x, (((0,), (0,)), ((), ())),
                                  preferred_element_type=jnp.float32)
    colsum = jnp.sum(x, axis=0, keepdims=True)
    s_ref[...] += jnp.concatenate([colsum] + [jnp.zeros_like(colsum)] * 7, axis=0)


def _moments(x):
    bn = x.shape[0]
    tp = 2048
    return pl.pallas_call(
        _mom_body,
        grid=(bn // tp,),
        in_specs=[pl.BlockSpec((tp, 64), lambda i: (i, 0))],
        out_specs=[pl.BlockSpec((64, 64), lambda i: (0, 0)),
                   pl.BlockSpec((8, 64), lambda i: (0, 0))],
        out_shape=[jax.ShapeDtypeStruct((64, 64), jnp.float32),
                   jax.ShapeDtypeStruct((8, 64), jnp.float32)],
        interpret=_IP,
    )(x)


def _final_body(x_ref, wt_ref, bo_ref, m_ref, s_ref, go_ref, beo_ref, o_ref, *,
                bn_total):
    bnf = jnp.float32(bn_total)
    meanx = s_ref[0:1] / bnf
    wt = wt_ref[...]
    mt = lax.dot_general(meanx, wt, (((1,), (0,)), ((), ())),
                         preferred_element_type=jnp.float32) + bo_ref[...]
    cov = m_ref[...] / bnf - lax.dot_general(meanx, meanx, (((0,), (0,)), ((), ())),
                                             preferred_element_type=jnp.float32)
    cw = lax.dot_general(cov, wt, (((1,), (0,)), ((), ())),
                         preferred_element_type=jnp.float32)
    var = jnp.sum(cw * wt, axis=0, keepdims=True)
    a = go_ref[...] / jnp.sqrt(var + 1e-5)
    d = beo_ref[...] - a * mt
    wa = wt * a
    o_ref[...] = jnp.maximum(
        lax.dot_general(x_ref[...], wa, (((1,), (0,)), ((), ())),
                        preferred_element_type=jnp.float32) + d, 0.0)


def _final(x, wt, bo, m, s, go, beo):
    bn = x.shape[0]
    tp = 2048
    return pl.pallas_call(
        functools.partial(_final_body, bn_total=bn),
        grid=(bn // tp,),
        in_specs=[pl.BlockSpec((tp, 64), lambda i: (i, 0)),
                  pl.BlockSpec((64, 128), lambda i: (0, 0)),
                  pl.BlockSpec((1, 128), lambda i: (0, 0)),
                  pl.BlockSpec((64, 64), lambda i: (0, 0)),
                  pl.BlockSpec((8, 64), lambda i: (0, 0)),
                  pl.BlockSpec((1, 128), lambda i: (0, 0)),
                  pl.BlockSpec((1, 128), lambda i: (0, 0))],
        out_specs=pl.BlockSpec((tp, 128), lambda i: (i, 0)),
        out_shape=jax.ShapeDtypeStruct((bn, 128), jnp.float32),
        interpret=_IP,
    )(x, wt, bo, m, s, go, beo)


def _edge_layer(x_pad, c_real, idx_flat, w, b, g, be):
    cp = x_pad.shape[1]
    wl = jnp.zeros((cp, 64), jnp.float32).at[:c_real].set(jnp.transpose(w[:, :c_real]))
    wr = jnp.zeros((cp, 64), jnp.float32).at[:c_real].set(jnp.transpose(w[:, c_real:]))
    feat = _sc_feat(x_pad, idx_flat)
    my, acc = _edge(feat, x_pad, wr, wl, b.reshape(1, 64))
    return _normalize(my, acc, g.reshape(1, 64), be.reshape(1, 64))


def kernel(point_features, point_coords, batch_size, W0, b0, g0, be0,
           W1, b1, g1, be1, Wo, bo, go, beo):
    del batch_size
    bn = point_features.shape[0]
    bsz = bn // _N

    coords = point_coords[:, 1:]
    xf = jnp.concatenate([coords, point_features], axis=-1)
    xf_pad = jnp.pad(xf, ((0, 0), (0, 32 - xf.shape[1])))
    pos_rows = jnp.pad(coords.reshape(bsz, _N, 3), ((0, 0), (0, 0), (0, 5)))
    idx1 = _knn_topk(pos_rows, jnp.transpose(pos_rows, (0, 2, 1)))
    idx1f = jnp.transpose(idx1, (0, 2, 1)).reshape(-1)
    x0 = _edge_layer(xf_pad, xf.shape[1], idx1f, W0, b0, g0, be0)

    x0r = x0.reshape(bsz, _N, 64)
    idx2 = _knn_topk(x0r, jnp.transpose(x0r, (0, 2, 1)))
    idx2f = jnp.transpose(idx2, (0, 2, 1)).reshape(-1)
    x1 = _edge_layer(x0, 64, idx2f, W1, b1, g1, be1)

    m, s = _moments(x1)
    return _final(x1, jnp.transpose(Wo), bo.reshape(1, 128), m, s,
                  go.reshape(1, 128), beo.reshape(1, 128))


# speedup vs baseline: 18.5916x; 1.3408x over previous
import functools

import jax
import jax.numpy as jnp
from jax import lax
from jax.experimental import pallas as pl
from jax.experimental.pallas import tpu as pltpu
from jax.experimental.pallas import tpu_sc as plsc

_N = 4096
_K = 16
_CW = 128
_NCH = _N // _CW
_T = 256
_TE = 128
_IP = False

_NC = 2
_NS = 16
_NW = _NC * _NS


def _knn_body(x_ref, xt_ref, out_ref):
    b = pl.program_id(0)
    xa = x_ref[0]
    xq = xt_ref[0]
    inner = lax.dot_general(xa, xq, (((1,), (0,)), ((), ())),
                            preferred_element_type=jnp.float32)
    sqa = jnp.sum(xa * xa, axis=1, keepdims=True)
    sqq = jnp.sum(xq * xq, axis=0, keepdims=True)
    d = (sqq - 2.0 * inner) + sqa
    d = jnp.maximum(d, 0.0) + 1.0
    bits = lax.bitcast_convert_type(d, jnp.int32)
    row = lax.broadcasted_iota(jnp.int32, d.shape, 0)
    keys = lax.bitcast_convert_type((bits & ~jnp.int32(_CW - 1)) | (row & (_CW - 1)),
                                    jnp.float32)
    cands = []
    for c in range(_NCH):
        kc = keys[c * _CW:(c + 1) * _CW, :]
        cur = jnp.zeros((1, kc.shape[1]), jnp.float32)
        for _ in range(_K):
            m = jnp.min(jnp.where(kc > cur, kc, jnp.inf), axis=0, keepdims=True)
            cands.append(m)
            cur = m
    ck = jnp.concatenate(cands, axis=0)
    cbits = lax.bitcast_convert_type(ck, jnp.int32)
    row2 = lax.broadcasted_iota(jnp.int32, ck.shape, 0)
    candcol = ((cbits & (_CW - 1)) + (row2 // _K) * _CW).astype(jnp.float32)
    for t in range(_K):
        m = jnp.min(ck, axis=0, keepdims=True)
        eq = ck == m
        j = jnp.min(jnp.where(eq, candcol, jnp.float32(1e9)), axis=0)
        out_ref[0, t, :] = j.astype(jnp.int32) + b * _N
        ck = jnp.where(candcol == j[None, :], jnp.inf, ck)


def _knn_topk(x, xt):
    bsz, n, c = x.shape
    return pl.pallas_call(
        _knn_body,
        grid=(bsz, n // _T),
        in_specs=[pl.BlockSpec((1, n, c), lambda b, r: (b, 0, 0)),
                  pl.BlockSpec((1, c, _T), lambda b, r: (b, 0, r))],
        out_specs=pl.BlockSpec((1, _K, _T), lambda b, r: (b, 0, r)),
        out_shape=jax.ShapeDtypeStruct((bsz, _K, n), jnp.int32),
        interpret=_IP,
    )(x, xt)


def _sc_feat(x, idxt):
    bn, c = x.shape
    pw = bn // _NW
    cp = 32
    nch = pw // cp
    mesh = plsc.VectorSubcoreMesh(core_axis_name="c", subcore_axis_name="s")

    @functools.partial(
        pl.kernel,
        out_type=jax.ShapeDtypeStruct((_K, bn, c), jnp.float32),
        mesh=mesh,
        scratch_types=[pltpu.VMEM((_K, cp), jnp.int32),
                       pltpu.VMEM((_K, cp), jnp.int32),
                       pltpu.VMEM((_K, cp, c), jnp.float32),
                       pltpu.VMEM((_K, cp, c), jnp.float32),
                       pltpu.SemaphoreType.DMA,
                       pltpu.SemaphoreType.DMA,
                       pltpu.SemaphoreType.DMA,
                       pltpu.SemaphoreType.DMA],
        compiler_params=pltpu.CompilerParams(use_tc_tiling_on_sc=False),
    )
    def run(x_hbm, idx_hbm, feat_hbm, idx0, idx1, buf0, buf1, gs0, gs1, ss0, ss1):
        wid = lax.axis_index("s") * _NC + lax.axis_index("c")
        base = wid * pw
        idxb = (idx0, idx1)
        bufs = (buf0, buf1)
        gsems = (gs0, gs1)
        ssems = (ss0, ss1)

        pltpu.sync_copy(idx_hbm.at[:, pl.ds(base, cp)], idx0)
        for k in range(_K):
            pltpu.async_copy(x_hbm.at[idx0.at[k]], buf0.at[k], gs0)

        def halfstep(ci, j):
            nci = ci + 1
            dst = pl.ds(base + ci * cp, cp)

            @pl.when(nci < nch)
            def _():
                pltpu.sync_copy(idx_hbm.at[:, pl.ds(base + nci * cp, cp)],
                                idxb[1 - j])

            for k in range(_K):
                pltpu.make_async_copy(x_hbm.at[idxb[j].at[k]], bufs[j].at[k],
                                      gsems[j]).wait()
            pltpu.async_copy(bufs[j], feat_hbm.at[:, dst, :], ssems[j])

            @pl.when(ci >= 1)
            def _():
                pltpu.make_async_copy(bufs[1 - j],
                                      feat_hbm.at[:, dst, :],
                                      ssems[1 - j]).wait()

            @pl.when(nci < nch)
            def _():
                for k in range(_K):
                    pltpu.async_copy(x_hbm.at[idxb[1 - j].at[k]],
                                     bufs[1 - j].at[k], gsems[1 - j])

        def pair(ci2, carry):
            halfstep(ci2 * 2, 0)
            halfstep(ci2 * 2 + 1, 1)
            return carry

        lax.fori_loop(0, nch // 2, pair, 0)
        pltpu.make_async_copy(buf1, feat_hbm.at[:, pl.ds(base, cp), :],
                              ss1).wait()

    return run(x, idxt)


def _edge_body(feat_ref, x_ref, wr_ref, wl_ref, b_ref, omax_ref, acc_ref):
    @pl.when(pl.program_id(0) == 0)
    def _():
        acc_ref[...] = jnp.zeros_like(acc_ref)

    x = x_ref[...]
    wr = wr_ref[...]
    z = lax.dot_general(x, wl_ref[...], (((1,), (0,)), ((), ())),
                        preferred_element_type=jnp.float32) + b_ref[...]
    mx = ya = qa = None
    for k in range(_K):
        diff = feat_ref[k] - x
        y = lax.dot_general(diff, wr, (((1,), (0,)), ((), ())),
                            preferred_element_type=jnp.float32) + z
        mx = y if k == 0 else jnp.maximum(mx, y)
        ya = y if k == 0 else ya + y
        qa = y * y if k == 0 else qa + y * y
    omax_ref[...] = mx
    acc_ref[...] += jnp.concatenate([jnp.sum(ya, axis=0, keepdims=True),
                                     jnp.sum(qa, axis=0, keepdims=True),
                                     jnp.zeros((6, 64), jnp.float32)], axis=0)


def _edge(feat, x, wr, wl, bvec):
    bn, c = x.shape
    return pl.pallas_call(
        _edge_body,
        grid=(bn // _TE,),
        in_specs=[pl.BlockSpec((_K, _TE, c), lambda i: (0, i, 0)),
                  pl.BlockSpec((_TE, c), lambda i: (i, 0)),
                  pl.BlockSpec((c, 64), lambda i: (0, 0)),
                  pl.BlockSpec((c, 64), lambda i: (0, 0)),
                  pl.BlockSpec((1, 64), lambda i: (0, 0))],
        out_specs=[pl.BlockSpec((_TE, 64), lambda i: (i, 0)),
                   pl.BlockSpec((8, 64), lambda i: (0, 0))],
        out_shape=[jax.ShapeDtypeStruct((bn, 64), jnp.float32),
                   jax.ShapeDtypeStruct((8, 64), jnp.float32)],
        interpret=_IP,
    )(feat, x, wr, wl, bvec)


def _norm_body(my_ref, acc_ref, g_ref, be_ref, o_ref, *, edges):
    acc = acc_ref[...]
    cnt = jnp.float32(edges)
    mean = acc[0:1] / cnt
    var = acc[1:2] / cnt - mean * mean
    yn = (my_ref[...] - mean) / jnp.sqrt(var + 1e-5)
    o_ref[...] = jnp.maximum(g_ref[...] * yn + be_ref[...], 0.0)


def _normalize(my, acc, g, be):
    bn = my.shape[0]
    tp = 2048
    return pl.pallas_call(
        functools.partial(_norm_body, edges=bn * _K),
        grid=(bn // tp,),
        in_specs=[pl.BlockSpec((tp, 64), lambda i: (i, 0)),
                  pl.BlockSpec((8, 64), lambda i: (0, 0)),
                  pl.BlockSpec((1, 64), lambda i: (0, 0)),
                  pl.BlockSpec((1, 64), lambda i: (0, 0))],
        out_specs=pl.BlockSpec((tp, 64), lambda i: (i, 0)),
        out_shape=jax.ShapeDtypeStruct((bn, 64), jnp.float32),
        interpret=_IP,
    )(my, acc, g, be)


def _mom_body(x_ref, m_ref, s_ref):
    @pl.when(pl.program_id(0) == 0)
    def _():
        m_ref[...] = jnp.zeros_like(m_ref)
        s_ref[...] = jnp.zeros_like(s_ref)

    x = x_ref[...]
    m_ref[...] += lax.dot_general(x, x, (((0,), (0,)), ((), ())),
                                  preferred_element_type=jnp.float32)
    colsum = jnp.sum(x, axis=0, keepdims=True)
    s_ref[...] += jnp.concatenate([colsum] + [jnp.zeros_like(colsum)] * 7, axis=0)


def _moments(x):
    bn = x.shape[0]
    tp = 2048
    return pl.pallas_call(
        _mom_body,
        grid=(bn // tp,),
        in_specs=[pl.BlockSpec((tp, 64), lambda i: (i, 0))],
        out_specs=[pl.BlockSpec((64, 64), lambda i: (0, 0)),
                   pl.BlockSpec((8, 64), lambda i: (0, 0))],
        out_shape=[jax.ShapeDtypeStruct((64, 64), jnp.float32),
                   jax.ShapeDtypeStruct((8, 64), jnp.float32)],
        interpret=_IP,
    )(x)


def _final_body(x_ref, wt_ref, bo_ref, m_ref, s_ref, go_ref, beo_ref, o_ref, *,
                bn_total):
    bnf = jnp.float32(bn_total)
    meanx = s_ref[0:1] / bnf
    wt = wt_ref[...]
    mt = lax.dot_general(meanx, wt, (((1,), (0,)), ((), ())),
                         preferred_element_type=jnp.float32) + bo_ref[...]
    cov = m_ref[...] / bnf - lax.dot_general(meanx, meanx, (((0,), (0,)), ((), ())),
                                             preferred_element_type=jnp.float32)
    cw = lax.dot_general(cov, wt, (((1,), (0,)), ((), ())),
                         preferred_element_type=jnp.float32)
    var = jnp.sum(cw * wt, axis=0, keepdims=True)
    a = go_ref[...] / jnp.sqrt(var + 1e-5)
    d = beo_ref[...] - a * mt
    wa = wt * a
    o_ref[...] = jnp.maximum(
        lax.dot_general(x_ref[...], wa, (((1,), (0,)), ((), ())),
                        preferred_element_type=jnp.float32) + d, 0.0)


def _final(x, wt, bo, m, s, go, beo):
    bn = x.shape[0]
    tp = 2048
    return pl.pallas_call(
        functools.partial(_final_body, bn_total=bn),
        grid=(bn // tp,),
        in_specs=[pl.BlockSpec((tp, 64), lambda i: (i, 0)),
                  pl.BlockSpec((64, 128), lambda i: (0, 0)),
                  pl.BlockSpec((1, 128), lambda i: (0, 0)),
                  pl.BlockSpec((64, 64), lambda i: (0, 0)),
                  pl.BlockSpec((8, 64), lambda i: (0, 0)),
                  pl.BlockSpec((1, 128), lambda i: (0, 0)),
                  pl.BlockSpec((1, 128), lambda i: (0, 0))],
        out_specs=pl.BlockSpec((tp, 128), lambda i: (i, 0)),
        out_shape=jax.ShapeDtypeStruct((bn, 128), jnp.float32),
        interpret=_IP,
    )(x, wt, bo, m, s, go, beo)


def _edge_layer(x_pad, c_real, idxt, w, b, g, be):
    cp = x_pad.shape[1]
    wl = jnp.zeros((cp, 64), jnp.float32).at[:c_real].set(jnp.transpose(w[:, :c_real]))
    wr = jnp.zeros((cp, 64), jnp.float32).at[:c_real].set(jnp.transpose(w[:, c_real:]))
    feat = _sc_feat(x_pad, idxt)
    my, acc = _edge(feat, x_pad, wr, wl, b.reshape(1, 64))
    return _normalize(my, acc, g.reshape(1, 64), be.reshape(1, 64))


def kernel(point_features, point_coords, batch_size, W0, b0, g0, be0,
           W1, b1, g1, be1, Wo, bo, go, beo):
    del batch_size
    bn = point_features.shape[0]
    bsz = bn // _N

    coords = point_coords[:, 1:]
    xf = jnp.concatenate([coords, point_features], axis=-1)
    xf_pad = jnp.pad(xf, ((0, 0), (0, 32 - xf.shape[1])))
    pos_rows = jnp.pad(coords.reshape(bsz, _N, 3), ((0, 0), (0, 0), (0, 5)))
    idx1 = _knn_topk(pos_rows, jnp.transpose(pos_rows, (0, 2, 1)))
    idx1t = jnp.transpose(idx1, (1, 0, 2)).reshape(_K, bn)
    x0 = _edge_layer(xf_pad, xf.shape[1], idx1t, W0, b0, g0, be0)

    x0r = x0.reshape(bsz, _N, 64)
    idx2 = _knn_topk(x0r, jnp.transpose(x0r, (0, 2, 1)))
    idx2t = jnp.transpose(idx2, (1, 0, 2)).reshape(_K, bn)
    x1 = _edge_layer(x0, 64, idx2t, W1, b1, g1, be1)

    m, s = _moments(x1)
    return _final(x1, jnp.transpose(Wo), bo.reshape(1, 128), m, s,
                  go.reshape(1, 128), beo.reshape(1, 128))
